# baseline (speedup 1.0000x reference)
import functools

import jax
import jax.numpy as jnp
from jax import lax
from jax.experimental import pallas as pl
from jax.experimental.pallas import tpu as pltpu
from jax.experimental.pallas import tpu_sc as plsc

_RBLK = 32768
_SC_BLOCKS = 12
_SLAB = 256


@functools.lru_cache(maxsize=None)
def _make_sc_transpose(V, D, A, NC, NS):
    NW = NC * NS
    cols_w = A // NW
    n_slab = cols_w // _SLAB
    n2 = n_slab // 2
    mesh = plsc.VectorSubcoreMesh(core_axis_name="c", subcore_axis_name="s")

    @functools.partial(
        pl.kernel,
        mesh=mesh,
        out_type=jax.ShapeDtypeStruct((A, D), jnp.float32),
        scratch_types=[
            pltpu.VMEM((D, _SLAB), jnp.float32),
            pltpu.VMEM((D, _SLAB), jnp.float32),
            pltpu.VMEM((_SLAB, D), jnp.float32),
            pltpu.VMEM((_SLAB, D), jnp.float32),
            pltpu.SemaphoreType.DMA,
            pltpu.SemaphoreType.DMA,
        ],
        compiler_params=pltpu.CompilerParams(needs_layout_passes=False),
    )
    def sc_t(tableT_hbm, out_hbm, in0, in1, t0, t1, sem_in, sem_out):
        wid = lax.axis_index("s") * NC + lax.axis_index("c")
        w_off = wid * cols_w
        lanes = lax.iota(jnp.int32, 16)

        def fire_fetch(j, buf):
            src = pl.multiple_of(w_off + j * _SLAB, 128)
            pltpu.async_copy(tableT_hbm.at[:, pl.ds(src, _SLAB)], buf, sem_in)

        def wait_fetch(buf):
            pltpu.make_async_copy(
                tableT_hbm.at[:, pl.ds(0, _SLAB)], buf, sem_in
            ).wait()

        def transpose_into(slab, tbuf):
            def rows16(r0, _):
                for k in range(16):
                    r = r0 * 16 + k
                    rr = jnp.full((16,), r, jnp.int32)
                    for q in range(D // 16):
                        vals = plsc.load_gather(slab, [lanes + q * 16, rr])
                        tbuf[r, pl.ds(q * 16, 16)] = vals
                return 0

            lax.fori_loop(0, _SLAB // 16, rows16, 0)

        def fire_write(j, tbuf):
            dst = pl.multiple_of(w_off + j * _SLAB, 128)
            pltpu.async_copy(tbuf, out_hbm.at[pl.ds(dst, _SLAB)], sem_out)

        def wait_write(tbuf):
            pltpu.make_async_copy(
                out_hbm.at[pl.ds(0, _SLAB)], tbuf, sem_out
            ).wait()

        fire_fetch(0, in0)
        fire_fetch(1, in1)
        fire_write(0, t0)
        fire_write(1, t1)

        def body(m, _):
            j0 = 2 * m
            for j, ibuf, tbuf in ((j0, in0, t0), (j0 + 1, in1, t1)):
                wait_write(tbuf)
                wait_fetch(ibuf)
                transpose_into(ibuf, tbuf)
                fire_write(j, tbuf)
                fire_fetch(jnp.minimum(j + 2, n_slab - 1), ibuf)
            return 0

        lax.fori_loop(0, n2, body, 0)
        wait_fetch(in0)
        wait_fetch(in1)
        wait_write(t0)
        wait_write(t1)

    return sc_t


def _transpose_body(xt_ref, o_ref):
    o_ref[...] = xt_ref[...].T


@functools.lru_cache(maxsize=None)
def _make_tc_transpose(V, D, A):
    rem = V - A
    grid = (rem + _RBLK - 1) // _RBLK
    off = A // _RBLK
    return pl.pallas_call(
        _transpose_body,
        grid=(grid,),
        in_specs=[pl.BlockSpec((D, _RBLK), lambda i: (0, off + i))],
        out_specs=pl.BlockSpec((_RBLK, D), lambda i: (i, 0)),
        out_shape=jax.ShapeDtypeStruct((rem, D), jnp.float32),
    )


@functools.lru_cache(maxsize=None)
def _make_gather(V, D, B, A, NC, NS):
    NW = NC * NS
    b_per_w = B // NW
    mesh = plsc.VectorSubcoreMesh(core_axis_name="c", subcore_axis_name="s")

    @functools.partial(
        pl.kernel,
        mesh=mesh,
        out_type=jax.ShapeDtypeStruct((B, D), jnp.float32),
        scratch_types=[
            pltpu.VMEM((b_per_w,), jnp.int32),
            pltpu.VMEM((b_per_w, D), jnp.float32),
            pltpu.SemaphoreType.DMA,
        ],
    )
    def gather(idx_hbm, lo_hbm, hi_hbm, out_hbm, idx_v, rows_v, sem):
        wid = lax.axis_index("s") * NC + lax.axis_index("c")
        base = wid * b_per_w
        lo3 = lo_hbm.reshape(A // 8, 8, D)
        hi3 = hi_hbm.reshape((V - A) // 8, 8, D)
        pltpu.sync_copy(idx_hbm.at[wid], idx_v)

        def body(g, _):
            vec = idx_v[pl.ds(g * 16, 16)]
            for k in range(16):
                tid = vec[k]
                hid = tid - A
                slot = g * 16 + k

                @pl.when(tid < A)
                def _():
                    pltpu.async_copy(
                        lo3.at[tid >> 3, tid & 7], rows_v.at[slot], sem
                    )

                @pl.when(tid >= A)
                def _():
                    pltpu.async_copy(
                        hi3.at[hid >> 3, hid & 7], rows_v.at[slot], sem
                    )

            return 0

        lax.fori_loop(0, b_per_w // 16, body, 0)
        pltpu.make_async_copy(
            lo_hbm.at[pl.ds(0, b_per_w)], rows_v, sem
        ).wait()
        pltpu.sync_copy(rows_v, out_hbm.at[pl.ds(base, b_per_w)])

    return gather


def _mlp_body(e_ref, w1_ref, b1_ref, w2_ref, b2_ref, o_ref):
    h = jnp.dot(e_ref[...], w1_ref[...], preferred_element_type=jnp.float32)
    h = jnp.maximum(h + b1_ref[...], 0.0)
    logit = jnp.sum(h * w2_ref[...], axis=1, keepdims=True) + b2_ref[...]
    o_ref[...] = 1.0 / (1.0 + jnp.exp(-logit))


@functools.lru_cache(maxsize=None)
def _make_mlp(B, H, F):
    BLK = 2048
    return pl.pallas_call(
        _mlp_body,
        grid=(B // BLK,),
        in_specs=[
            pl.BlockSpec((BLK, H), lambda i: (i, 0)),
            pl.BlockSpec((H, F), lambda i: (0, 0)),
            pl.BlockSpec((1, F), lambda i: (0, 0)),
            pl.BlockSpec((1, F), lambda i: (0, 0)),
            pl.BlockSpec((1, 1), lambda i: (0, 0)),
        ],
        out_specs=pl.BlockSpec((BLK, 1), lambda i: (i, 0)),
        out_shape=jax.ShapeDtypeStruct((B, 1), jnp.float32),
    )


def kernel(tool_token, table, W1, b1, W2, b2):
    B = tool_token.shape[0]
    V, D = table.shape
    H, F = W1.shape
    info = plsc.get_sparse_core_info()
    NC, NS = info.num_cores, info.num_subcores
    NW = NC * NS
    b_per_w = B // NW
    A = _SC_BLOCKS * _RBLK
    idx = tool_token.astype(jnp.int32).reshape(NW, b_per_w)
    tT = table.T
    lo = _make_sc_transpose(V, D, A, NC, NS)(tT)
    hi = _make_tc_transpose(V, D, A)(tT)
    emb = _make_gather(V, D, B, A, NC, NS)(idx, lo, hi)
    out = _make_mlp(B, H, F)(
        emb,
        W1,
        b1.reshape(1, F),
        W2.reshape(1, F),
        b2.reshape(1, 1),
    )
    return out


# speedup vs baseline: 2.2855x; 2.2855x over previous
import functools

import jax
import jax.numpy as jnp
from jax import lax
from jax.experimental import pallas as pl
from jax.experimental.pallas import tpu as pltpu
from jax.experimental.pallas import tpu_sc as plsc


def _transpose_body(xt_ref, o_ref):
    o_ref[...] = xt_ref[...].T


@functools.lru_cache(maxsize=None)
def _make_transpose(V, D):
    RBLK = 32768
    grid = (V + RBLK - 1) // RBLK
    return pl.pallas_call(
        _transpose_body,
        grid=(grid,),
        in_specs=[pl.BlockSpec((D, RBLK), lambda i: (0, i))],
        out_specs=pl.BlockSpec((RBLK, D), lambda i: (i, 0)),
        out_shape=jax.ShapeDtypeStruct((V, D), jnp.float32),
    )


@functools.lru_cache(maxsize=None)
def _make_gather(V, D, B, NC, NS):
    NW = NC * NS
    b_per_w = B // NW
    mesh = plsc.VectorSubcoreMesh(core_axis_name="c", subcore_axis_name="s")

    @functools.partial(
        pl.kernel,
        mesh=mesh,
        out_type=jax.ShapeDtypeStruct((B, D), jnp.float32),
        scratch_types=[
            pltpu.VMEM((b_per_w,), jnp.int32),
            pltpu.VMEM((b_per_w, D), jnp.float32),
            pltpu.SemaphoreType.DMA,
        ],
    )
    def gather(idx_hbm, table_hbm, out_hbm, idx_v, rows_v, sem):
        wid = lax.axis_index("s") * NC + lax.axis_index("c")
        base = wid * b_per_w
        table3 = table_hbm.reshape(V // 8, 8, D)
        pltpu.sync_copy(idx_hbm.at[wid], idx_v)

        def body(g, _):
            vec = idx_v[pl.ds(g * 16, 16)]
            for k in range(16):
                tid = vec[k]
                pltpu.async_copy(
                    table3.at[tid >> 3, tid & 7],
                    rows_v.at[g * 16 + k],
                    sem,
                )
            return 0

        lax.fori_loop(0, b_per_w // 16, body, 0)
        pltpu.make_async_copy(
            table_hbm.at[pl.ds(0, b_per_w)], rows_v, sem
        ).wait()
        pltpu.sync_copy(rows_v, out_hbm.at[pl.ds(base, b_per_w)])

    return gather


def _mlp_body(e_ref, w1_ref, b1_ref, w2_ref, b2_ref, o_ref):
    h = jnp.dot(e_ref[...], w1_ref[...], preferred_element_type=jnp.float32)
    h = jnp.maximum(h + b1_ref[...], 0.0)
    logit = jnp.sum(h * w2_ref[...], axis=1, keepdims=True) + b2_ref[...]
    o_ref[...] = 1.0 / (1.0 + jnp.exp(-logit))


@functools.lru_cache(maxsize=None)
def _make_mlp(B, H, F):
    BLK = 2048
    return pl.pallas_call(
        _mlp_body,
        grid=(B // BLK,),
        in_specs=[
            pl.BlockSpec((BLK, H), lambda i: (i, 0)),
            pl.BlockSpec((H, F), lambda i: (0, 0)),
            pl.BlockSpec((1, F), lambda i: (0, 0)),
            pl.BlockSpec((1, F), lambda i: (0, 0)),
            pl.BlockSpec((1, 1), lambda i: (0, 0)),
        ],
        out_specs=pl.BlockSpec((BLK, 1), lambda i: (i, 0)),
        out_shape=jax.ShapeDtypeStruct((B, 1), jnp.float32),
    )


def kernel(tool_token, table, W1, b1, W2, b2):
    B = tool_token.shape[0]
    V, D = table.shape
    H, F = W1.shape
    info = plsc.get_sparse_core_info()
    NC, NS = info.num_cores, info.num_subcores
    NW = NC * NS
    b_per_w = B // NW
    idx = tool_token.astype(jnp.int32).reshape(NW, b_per_w)
    table_rm = _make_transpose(V, D)(table.T)
    emb = _make_gather(V, D, B, NC, NS)(idx, table_rm)
    out = _make_mlp(B, H, F)(
        emb,
        W1,
        b1.reshape(1, F),
        W2.reshape(1, F),
        b2.reshape(1, 1),
    )
    return out


# speedup vs baseline: 2.3236x; 1.0167x over previous
import functools

import jax
import jax.numpy as jnp
from jax import lax
from jax.experimental import pallas as pl
from jax.experimental.pallas import tpu as pltpu
from jax.experimental.pallas import tpu_sc as plsc


def _transpose_body(xt_ref, o_ref):
    o_ref[...] = xt_ref[...].T


@functools.lru_cache(maxsize=None)
def _make_transpose(V, D):
    RBLK = 32768
    grid = (V + RBLK - 1) // RBLK
    return pl.pallas_call(
        _transpose_body,
        grid=(grid,),
        in_specs=[pl.BlockSpec((D, RBLK), lambda i: (0, i))],
        out_specs=pl.BlockSpec((RBLK, D), lambda i: (i, 0)),
        out_shape=jax.ShapeDtypeStruct((V, D), jnp.float32),
    )


@functools.lru_cache(maxsize=None)
def _make_gather(V, D, B, NC, NS):
    NW = NC * NS
    b_per_w = B // NW
    mesh = plsc.VectorSubcoreMesh(core_axis_name="c", subcore_axis_name="s")

    @functools.partial(
        pl.kernel,
        mesh=mesh,
        out_type=jax.ShapeDtypeStruct((B, D), jnp.float32),
        scratch_types=[
            pltpu.VMEM((b_per_w,), jnp.int32),
            pltpu.VMEM((b_per_w, D), jnp.float32),
            pltpu.SemaphoreType.DMA,
        ],
    )
    def gather(idx_hbm, table_hbm, out_hbm, idx_v, rows_v, sem):
        wid = lax.axis_index("s") * NC + lax.axis_index("c")
        base = wid * b_per_w
        table3 = table_hbm.reshape(V // 8, 8, D)
        pltpu.sync_copy(idx_hbm.at[wid], idx_v)

        def body(g, _):
            vec = idx_v[pl.ds(g * 16, 16)]
            for k in range(16):
                tid = vec[k]
                pltpu.async_copy(
                    table3.at[tid >> 3, tid & 7],
                    rows_v.at[g * 16 + k],
                    sem,
                )
            return 0

        lax.fori_loop(0, b_per_w // 16, body, 0)
        pltpu.make_async_copy(
            table_hbm.at[pl.ds(0, b_per_w)], rows_v, sem
        ).wait()
        pltpu.sync_copy(rows_v, out_hbm.at[pl.ds(base, b_per_w)])

    return gather


def _mlp_body(e_ref, w1_ref, b1_ref, w2_ref, b2_ref, o_ref):
    h = jnp.dot(e_ref[...], w1_ref[...], preferred_element_type=jnp.float32)
    h = jnp.maximum(h + b1_ref[...], 0.0)
    logit = jnp.sum(h * w2_ref[...], axis=1, keepdims=True) + b2_ref[...]
    o_ref[...] = 1.0 / (1.0 + jnp.exp(-logit))


@functools.lru_cache(maxsize=None)
def _make_mlp(B, H, F):
    BLK = 8192
    return pl.pallas_call(
        _mlp_body,
        grid=(B // BLK,),
        in_specs=[
            pl.BlockSpec((BLK, H), lambda i: (i, 0)),
            pl.BlockSpec((H, F), lambda i: (0, 0)),
            pl.BlockSpec((1, F), lambda i: (0, 0)),
            pl.BlockSpec((1, F), lambda i: (0, 0)),
            pl.BlockSpec((1, 1), lambda i: (0, 0)),
        ],
        out_specs=pl.BlockSpec((BLK, 1), lambda i: (i, 0)),
        out_shape=jax.ShapeDtypeStruct((B, 1), jnp.float32),
    )


def kernel(tool_token, table, W1, b1, W2, b2):
    B = tool_token.shape[0]
    V, D = table.shape
    H, F = W1.shape
    info = plsc.get_sparse_core_info()
    NC, NS = info.num_cores, info.num_subcores
    NW = NC * NS
    b_per_w = B // NW
    idx = tool_token.astype(jnp.int32).reshape(NW, b_per_w)
    table_rm = _make_transpose(V, D)(table.T)
    emb = _make_gather(V, D, B, NC, NS)(idx, table_rm)
    out = _make_mlp(B, H, F)(
        emb,
        W1,
        b1.reshape(1, F),
        W2.reshape(1, F),
        b2.reshape(1, 1),
    )
    return out
